# baseline (speedup 1.0000x reference)
import jax
import jax.numpy as jnp
from jax import lax
from jax.experimental import pallas as pl
from jax.experimental.pallas import tpu as pltpu
from jax.experimental.pallas import tpu_sc as plsc

H = 384
W = 384
HW = H * W
NFRAMES = 16
CR = 4
CHUNK = CR * W
NCHUNK = H // CR
RING = 92
RB = RING * W
MODM = 45591
MAGIC = 8388608.0
PRIME_ROWS = 52


def _fire_body(x_hbm, flow_hbm, out_hbm, *sc):
    ring = sc[0:3]
    fx_b = sc[3:5]
    fy_b = sc[5:7]
    xc_b = (sc[7:10], sc[10:13])
    out_b = (sc[13:16], sc[16:19])
    fixidx = sc[19]
    fix_v = sc[20:23]
    sem_slide, sem_f, sem_xc, sem_st0, sem_st1, sem_fix = sc[23:29]
    sem_st = (sem_st0, sem_st1)

    cid = lax.axis_index("c")
    sid = lax.axis_index("s")
    wid = sid * 2 + cid
    tm1 = wid // 2
    d = wid % 2
    f = jnp.where(d == 0, tm1, 31 - tm1)
    src_t = jnp.where(d == 0, tm1 + 2, tm1)
    cur_t = tm1 + 1
    out_frame = tm1 * 6 + d * 3
    src_ch_base = [(src_t * 3 + ch) * HW for ch in range(3)]
    cur_ch_base = [(cur_t * 3 + ch) * HW for ch in range(3)]
    out_ch_base = [(out_frame + ch) * HW for ch in range(3)]
    fx_off = 2 * f * HW
    fy_off = (2 * f + 1) * HW

    iota_f = lax.iota(jnp.int32, 16).astype(jnp.float32)

    def issue_slide(c):
        c4 = c * CR
        row = jnp.where(c4 + 48 <= H - CR, c4 + 48, H - CR)
        slot = lax.rem(row, RING)
        for ch in range(3):
            pltpu.async_copy(
                x_hbm.at[pl.ds(src_ch_base[ch] + row * W, CHUNK)],
                ring[ch].at[pl.ds(slot * W, CHUNK)],
                sem_slide,
            )

    def wait_slide():
        for ch in range(3):
            pltpu.make_async_copy(
                x_hbm.at[pl.ds(0, CHUNK)], ring[ch].at[pl.ds(0, CHUNK)], sem_slide
            ).wait()

    def issue_prefetch(c, pp):
        c4 = c * CR
        b = jnp.where(c4 <= H - CR, c4, H - CR) * W
        pltpu.async_copy(flow_hbm.at[pl.ds(fx_off + b, CHUNK)], fx_b[pp], sem_f)
        pltpu.async_copy(flow_hbm.at[pl.ds(fy_off + b, CHUNK)], fy_b[pp], sem_f)
        for ch in range(3):
            pltpu.async_copy(
                x_hbm.at[pl.ds(cur_ch_base[ch] + b, CHUNK)], xc_b[pp][ch], sem_xc
            )

    def wait_prefetch(pp):
        pltpu.make_async_copy(
            flow_hbm.at[pl.ds(0, CHUNK)], fx_b[pp], sem_f
        ).wait()
        pltpu.make_async_copy(
            flow_hbm.at[pl.ds(0, CHUNK)], fy_b[pp], sem_f
        ).wait()
        for ch in range(3):
            pltpu.make_async_copy(
                x_hbm.at[pl.ds(0, CHUNK)], xc_b[pp][ch], sem_xc
            ).wait()

    def do_chunk(c, pp, first=False):
        c4 = c * CR
        lo = jnp.where(c4 >= 40, c4 - 40, 0)
        hi = jnp.where(c4 + 47 <= H - 1, c4 + 47, H - 1)
        lo_f = lo.astype(jnp.float32)
        hi_f = hi.astype(jnp.float32)
        row0_f = c4.astype(jnp.float32)

        issue_slide(c)
        issue_prefetch(c + 1, (pp + 1) % 2)

        if not first:

            @pl.when(c >= 2)
            def _():
                for ch in range(3):
                    pltpu.make_async_copy(
                        out_b[pp][ch],
                        out_hbm.at[pl.ds(out_ch_base[ch] + c4 * W, CHUNK)],
                        sem_st[pp],
                    ).wait()

        def row_body(r, _):
            row_f = row0_f + r.astype(jnp.float32)
            roff = r * W
            for k in range(W // 16):
                off = roff + k * 16
                fxv = fx_b[pp][pl.ds(off, 16)]
                fyv = fy_b[pp][pl.ds(off, 16)]
                rx = jnp.minimum(jnp.maximum(fxv + row_f, 0.0), float(H - 1))
                rx = (rx + MAGIC) - MAGIC
                colv = iota_f + float(k * 16)
                ry = jnp.minimum(jnp.maximum(fyv + colv, 0.0), float(W - 1))
                ry = (ry + MAGIC) - MAGIC
                ryi = ry.astype(jnp.int32)
                rxc = jnp.minimum(jnp.maximum(rx, lo_f), hi_f).astype(jnp.int32)
                q = (rxc * MODM) >> 22
                addr = (rxc - q * RING) * W + ryi
                xcv = []
                for ch in range(3):
                    g = plsc.load_gather(ring[ch], [addr])
                    xv = xc_b[pp][ch][pl.ds(off, 16)]
                    xcv.append(xv)
                    out_b[pp][ch][pl.ds(off, 16)] = xv - g
                out_of_win = jnp.logical_or(rx < lo_f, rx > hi_f)

                @pl.when(jnp.any(out_of_win))
                def _():
                    flat = (rx * float(W) + ry).astype(jnp.int32)
                    fixidx[...] = flat
                    for ch in range(3):
                        pltpu.async_copy(
                            x_hbm.at[pl.ds(src_ch_base[ch], HW)].at[fixidx],
                            fix_v[ch],
                            sem_fix,
                        )
                    for ch in range(3):
                        pltpu.make_async_copy(
                            x_hbm.at[pl.ds(0, 16)], fix_v[ch], sem_fix
                        ).wait()
                        out_b[pp][ch][pl.ds(off, 16)] = (
                            xcv[ch] - fix_v[ch][...]
                        )

            return 0

        lax.fori_loop(0, CR, row_body, 0)


        for ch in range(3):
            pltpu.async_copy(
                out_b[pp][ch],
                out_hbm.at[pl.ds(out_ch_base[ch] + c4 * W, CHUNK)],
                sem_st[pp],
            )

    for ch in range(3):
        pltpu.async_copy(
            x_hbm.at[pl.ds(src_ch_base[ch], PRIME_ROWS * W)],
            ring[ch].at[pl.ds(0, PRIME_ROWS * W)],
            sem_slide,
        )
    issue_prefetch(jnp.int32(0), 0)
    for ch in range(3):
        pltpu.make_async_copy(
            x_hbm.at[pl.ds(0, PRIME_ROWS * W)],
            ring[ch].at[pl.ds(0, PRIME_ROWS * W)],
            sem_slide,
        ).wait()
    wait_prefetch(0)

    do_chunk(jnp.int32(0), 0, first=True)

    def super_body(s, _):
        c = 2 * s + 1
        wait_slide()
        wait_prefetch(1)
        do_chunk(c, 1)
        wait_slide()
        wait_prefetch(0)
        do_chunk(c + 1, 0)
        return 0

    lax.fori_loop(0, (NCHUNK - 2) // 2, super_body, 0)

    wait_slide()
    wait_prefetch(1)
    do_chunk(jnp.int32(NCHUNK - 1), 1)
    wait_slide()
    wait_prefetch(0)

    for pp in (0, 1):
        c4 = (NCHUNK - 2 + pp) * CR
        for ch in range(3):
            pltpu.make_async_copy(
                out_b[pp][ch],
                out_hbm.at[pl.ds(out_ch_base[ch] + c4 * W, CHUNK)],
                sem_st[pp],
            ).wait()


@jax.jit
def kernel(x, flow):
    x_flat = x.reshape(-1)
    flow_flat = flow.reshape(-1)

    mesh = plsc.VectorSubcoreMesh(core_axis_name="c", subcore_axis_name="s")
    out = pl.kernel(
        _fire_body,
        out_type=jax.ShapeDtypeStruct((NFRAMES * 6 * HW,), jnp.float32),
        mesh=mesh,
        compiler_params=pltpu.CompilerParams(needs_layout_passes=False),
        scratch_types=[
            pltpu.VMEM((RB,), jnp.float32),
            pltpu.VMEM((RB,), jnp.float32),
            pltpu.VMEM((RB,), jnp.float32),
            pltpu.VMEM((CHUNK,), jnp.float32),
            pltpu.VMEM((CHUNK,), jnp.float32),
            pltpu.VMEM((CHUNK,), jnp.float32),
            pltpu.VMEM((CHUNK,), jnp.float32),
            pltpu.VMEM((CHUNK,), jnp.float32),
            pltpu.VMEM((CHUNK,), jnp.float32),
            pltpu.VMEM((CHUNK,), jnp.float32),
            pltpu.VMEM((CHUNK,), jnp.float32),
            pltpu.VMEM((CHUNK,), jnp.float32),
            pltpu.VMEM((CHUNK,), jnp.float32),
            pltpu.VMEM((CHUNK,), jnp.float32),
            pltpu.VMEM((CHUNK,), jnp.float32),
            pltpu.VMEM((CHUNK,), jnp.float32),
            pltpu.VMEM((CHUNK,), jnp.float32),
            pltpu.VMEM((CHUNK,), jnp.float32),
            pltpu.VMEM((CHUNK,), jnp.float32),
            pltpu.VMEM((16,), jnp.int32),
            pltpu.VMEM((16,), jnp.float32),
            pltpu.VMEM((16,), jnp.float32),
            pltpu.VMEM((16,), jnp.float32),
            pltpu.SemaphoreType.DMA,
            pltpu.SemaphoreType.DMA,
            pltpu.SemaphoreType.DMA,
            pltpu.SemaphoreType.DMA,
            pltpu.SemaphoreType.DMA,
            pltpu.SemaphoreType.DMA,
        ],
    )(x_flat, flow_flat)
    return out.reshape(NFRAMES, 6, H, W)


# speedup vs baseline: 3.8441x; 3.8441x over previous
import jax
import jax.numpy as jnp
from jax import lax
from jax.experimental import pallas as pl
from jax.experimental.pallas import tpu as pltpu
from jax.experimental.pallas import tpu_sc as plsc

H = 384
W = 384
HW = H * W
NFRAMES = 16
CR = 4
CHUNK = CR * W
NCHUNK = H // CR
RING = 92
RB = RING * W
MODM = 45591
MAGIC = 8388608.0
PRIME_ROWS = 52


def _fire_body(x_hbm, flow_hbm, out_hbm, *sc):
    ring = sc[0:3]
    fx_b = sc[3:5]
    fy_b = sc[5:7]
    xc_b = (sc[7:10], sc[10:13])
    out_b = (sc[13:16], sc[16:19])
    fixidx = sc[19]
    sem_slide, sem_f, sem_xc, sem_st0, sem_st1, sem_fix = sc[20:26]
    sem_st = (sem_st0, sem_st1)

    cid = lax.axis_index("c")
    sid = lax.axis_index("s")
    wid = sid * 2 + cid
    tm1 = wid // 2
    d = wid % 2
    f = jnp.where(d == 0, tm1, 31 - tm1)
    src_t = jnp.where(d == 0, tm1 + 2, tm1)
    cur_t = tm1 + 1
    out_frame = tm1 * 6 + d * 3
    src_ch_base = [(src_t * 3 + ch) * HW for ch in range(3)]
    cur_ch_base = [(cur_t * 3 + ch) * HW for ch in range(3)]
    out_ch_base = [(out_frame + ch) * HW for ch in range(3)]
    fx_off = 2 * f * HW
    fy_off = (2 * f + 1) * HW

    iota_f = lax.iota(jnp.int32, 16).astype(jnp.float32)

    def issue_slide(c):
        c4 = c * CR
        row = jnp.where(c4 + 48 <= H - CR, c4 + 48, H - CR)
        slot = lax.rem(row, RING)
        for ch in range(3):
            pltpu.async_copy(
                x_hbm.at[pl.ds(src_ch_base[ch] + row * W, CHUNK)],
                ring[ch].at[pl.ds(slot * W, CHUNK)],
                sem_slide,
            )

    def wait_slide():
        for ch in range(3):
            pltpu.make_async_copy(
                x_hbm.at[pl.ds(0, CHUNK)], ring[ch].at[pl.ds(0, CHUNK)], sem_slide
            ).wait()

    def issue_prefetch(c, pp):
        c4 = c * CR
        b = jnp.where(c4 <= H - CR, c4, H - CR) * W
        pltpu.async_copy(flow_hbm.at[pl.ds(fx_off + b, CHUNK)], fx_b[pp], sem_f)
        pltpu.async_copy(flow_hbm.at[pl.ds(fy_off + b, CHUNK)], fy_b[pp], sem_f)
        for ch in range(3):
            pltpu.async_copy(
                x_hbm.at[pl.ds(cur_ch_base[ch] + b, CHUNK)], xc_b[pp][ch], sem_xc
            )

    def wait_prefetch(pp):
        pltpu.make_async_copy(
            flow_hbm.at[pl.ds(0, CHUNK)], fx_b[pp], sem_f
        ).wait()
        pltpu.make_async_copy(
            flow_hbm.at[pl.ds(0, CHUNK)], fy_b[pp], sem_f
        ).wait()
        for ch in range(3):
            pltpu.make_async_copy(
                x_hbm.at[pl.ds(0, CHUNK)], xc_b[pp][ch], sem_xc
            ).wait()

    def do_chunk(c, pp, first=False):
        c4 = c * CR
        lo = jnp.where(c4 >= 40, c4 - 40, 0)
        hi = jnp.where(c4 + 47 <= H - 1, c4 + 47, H - 1)
        lo_f = lo.astype(jnp.float32)
        hi_f = hi.astype(jnp.float32)
        row0_f = c4.astype(jnp.float32)

        issue_slide(c)
        issue_prefetch(c + 1, (pp + 1) % 2)

        if not first:

            @pl.when(c >= 2)
            def _():
                for ch in range(3):
                    pltpu.make_async_copy(
                        out_b[pp][ch],
                        out_hbm.at[pl.ds(out_ch_base[ch] + c4 * W, CHUNK)],
                        sem_st[pp],
                    ).wait()

        def row_body(r, _):
            row_f = row0_f + r.astype(jnp.float32)
            roff = r * W
            rx_mn = None
            rx_mx = None
            for k in range(W // 16):
                off = roff + k * 16
                fxv = fx_b[pp][pl.ds(off, 16)]
                fyv = fy_b[pp][pl.ds(off, 16)]
                rx = jnp.minimum(jnp.maximum(fxv + row_f, 0.0), float(H - 1))
                rx = (rx + MAGIC) - MAGIC
                colv = iota_f + float(k * 16)
                ry = jnp.minimum(jnp.maximum(fyv + colv, 0.0), float(W - 1))
                ry = (ry + MAGIC) - MAGIC
                ryi = ry.astype(jnp.int32)
                rxc = jnp.minimum(jnp.maximum(rx, lo_f), hi_f).astype(jnp.int32)
                q = (rxc * MODM) >> 22
                addr = (rxc - q * RING) * W + ryi
                rx_mn = rx if rx_mn is None else jnp.minimum(rx_mn, rx)
                rx_mx = rx if rx_mx is None else jnp.maximum(rx_mx, rx)
                for ch in range(3):
                    g = plsc.load_gather(ring[ch], [addr])
                    xv = xc_b[pp][ch][pl.ds(off, 16)]
                    out_b[pp][ch][pl.ds(off, 16)] = xv - g

            row_bad = jnp.logical_or(
                jnp.min(rx_mn) < lo_f, jnp.max(rx_mx) > hi_f
            )

            @pl.when(row_bad)
            def _():
                def fix_k(kk, _):
                    o = roff + kk * 16
                    fxv = fx_b[pp][pl.ds(o, 16)]
                    fyv = fy_b[pp][pl.ds(o, 16)]
                    rx = jnp.minimum(jnp.maximum(fxv + row_f, 0.0), float(H - 1))
                    rx = (rx + MAGIC) - MAGIC
                    colv = iota_f + (kk * 16).astype(jnp.float32)
                    ry = jnp.minimum(jnp.maximum(fyv + colv, 0.0), float(W - 1))
                    ry = (ry + MAGIC) - MAGIC
                    fixidx[pl.ds(kk * 16, 16)] = (rx * float(W) + ry).astype(
                        jnp.int32
                    )
                    return 0

                lax.fori_loop(0, W // 16, fix_k, 0)
                for ch in range(3):
                    pltpu.async_copy(
                        x_hbm.at[pl.ds(src_ch_base[ch], HW)].at[fixidx],
                        out_b[pp][ch].at[pl.ds(roff, W)],
                        sem_fix,
                    )
                for ch in range(3):
                    pltpu.make_async_copy(
                        x_hbm.at[pl.ds(0, W)],
                        out_b[pp][ch].at[pl.ds(roff, W)],
                        sem_fix,
                    ).wait()

                def fix_sub(kk, _):
                    o = roff + kk * 16
                    for ch in range(3):
                        out_b[pp][ch][pl.ds(o, 16)] = (
                            xc_b[pp][ch][pl.ds(o, 16)]
                            - out_b[pp][ch][pl.ds(o, 16)]
                        )
                    return 0

                lax.fori_loop(0, W // 16, fix_sub, 0)

            return 0

        lax.fori_loop(0, CR, row_body, 0)


        for ch in range(3):
            pltpu.async_copy(
                out_b[pp][ch],
                out_hbm.at[pl.ds(out_ch_base[ch] + c4 * W, CHUNK)],
                sem_st[pp],
            )

    for ch in range(3):
        pltpu.async_copy(
            x_hbm.at[pl.ds(src_ch_base[ch], PRIME_ROWS * W)],
            ring[ch].at[pl.ds(0, PRIME_ROWS * W)],
            sem_slide,
        )
    issue_prefetch(jnp.int32(0), 0)
    for ch in range(3):
        pltpu.make_async_copy(
            x_hbm.at[pl.ds(0, PRIME_ROWS * W)],
            ring[ch].at[pl.ds(0, PRIME_ROWS * W)],
            sem_slide,
        ).wait()
    wait_prefetch(0)

    do_chunk(jnp.int32(0), 0, first=True)

    def super_body(s, _):
        c = 2 * s + 1
        wait_slide()
        wait_prefetch(1)
        do_chunk(c, 1)
        wait_slide()
        wait_prefetch(0)
        do_chunk(c + 1, 0)
        return 0

    lax.fori_loop(0, (NCHUNK - 2) // 2, super_body, 0)

    wait_slide()
    wait_prefetch(1)
    do_chunk(jnp.int32(NCHUNK - 1), 1)
    wait_slide()
    wait_prefetch(0)

    for pp in (0, 1):
        c4 = (NCHUNK - 2 + pp) * CR
        for ch in range(3):
            pltpu.make_async_copy(
                out_b[pp][ch],
                out_hbm.at[pl.ds(out_ch_base[ch] + c4 * W, CHUNK)],
                sem_st[pp],
            ).wait()


@jax.jit
def kernel(x, flow):
    x_flat = x.reshape(-1)
    flow_flat = flow.reshape(-1)

    mesh = plsc.VectorSubcoreMesh(core_axis_name="c", subcore_axis_name="s")
    out = pl.kernel(
        _fire_body,
        out_type=jax.ShapeDtypeStruct((NFRAMES * 6 * HW,), jnp.float32),
        mesh=mesh,
        compiler_params=pltpu.CompilerParams(needs_layout_passes=False),
        scratch_types=[
            pltpu.VMEM((RB,), jnp.float32),
            pltpu.VMEM((RB,), jnp.float32),
            pltpu.VMEM((RB,), jnp.float32),
            pltpu.VMEM((CHUNK,), jnp.float32),
            pltpu.VMEM((CHUNK,), jnp.float32),
            pltpu.VMEM((CHUNK,), jnp.float32),
            pltpu.VMEM((CHUNK,), jnp.float32),
            pltpu.VMEM((CHUNK,), jnp.float32),
            pltpu.VMEM((CHUNK,), jnp.float32),
            pltpu.VMEM((CHUNK,), jnp.float32),
            pltpu.VMEM((CHUNK,), jnp.float32),
            pltpu.VMEM((CHUNK,), jnp.float32),
            pltpu.VMEM((CHUNK,), jnp.float32),
            pltpu.VMEM((CHUNK,), jnp.float32),
            pltpu.VMEM((CHUNK,), jnp.float32),
            pltpu.VMEM((CHUNK,), jnp.float32),
            pltpu.VMEM((CHUNK,), jnp.float32),
            pltpu.VMEM((CHUNK,), jnp.float32),
            pltpu.VMEM((CHUNK,), jnp.float32),
            pltpu.VMEM((W,), jnp.int32),
            pltpu.SemaphoreType.DMA,
            pltpu.SemaphoreType.DMA,
            pltpu.SemaphoreType.DMA,
            pltpu.SemaphoreType.DMA,
            pltpu.SemaphoreType.DMA,
            pltpu.SemaphoreType.DMA,
        ],
    )(x_flat, flow_flat)
    return out.reshape(NFRAMES, 6, H, W)


# speedup vs baseline: 4.8114x; 1.2517x over previous
import jax
import jax.numpy as jnp
from jax import lax
from jax.experimental import pallas as pl
from jax.experimental.pallas import tpu as pltpu
from jax.experimental.pallas import tpu_sc as plsc

H = 384
W = 384
HW = H * W
NFRAMES = 16
CR = 4
CHUNK = CR * W
NCHUNK = H // CR
RING = 92
RB = RING * W
MODM = 45591
MAGIC = 8388608.0
PRIME_ROWS = 52


def _fire_body(x_hbm, flow_hbm, out_hbm, *sc):
    ring = sc[0:3]
    fx_b = sc[3:5]
    fy_b = sc[5:7]
    xc_b = (sc[7:10], sc[10:13])
    out_b = (sc[13:16], sc[16:19])
    fixidx = sc[19]
    sem_slide, sem_f, sem_xc, sem_st0, sem_st1, sem_fix = sc[20:26]
    sem_st = (sem_st0, sem_st1)

    cid = lax.axis_index("c")
    sid = lax.axis_index("s")
    wid = sid * 2 + cid
    tm1 = wid // 2
    d = wid % 2
    f = jnp.where(d == 0, tm1, 31 - tm1)
    src_t = jnp.where(d == 0, tm1 + 2, tm1)
    cur_t = tm1 + 1
    out_frame = tm1 * 6 + d * 3
    src_ch_base = [(src_t * 3 + ch) * HW for ch in range(3)]
    cur_ch_base = [(cur_t * 3 + ch) * HW for ch in range(3)]
    out_ch_base = [(out_frame + ch) * HW for ch in range(3)]
    fx_off = 2 * f * HW
    fy_off = (2 * f + 1) * HW

    iota_f = lax.iota(jnp.int32, 16).astype(jnp.float32)

    def issue_slide(c):
        c4 = c * CR
        row = jnp.where(c4 + 48 <= H - CR, c4 + 48, H - CR)
        slot = lax.rem(row, RING)
        for ch in range(3):
            pltpu.async_copy(
                x_hbm.at[pl.ds(src_ch_base[ch] + row * W, CHUNK)],
                ring[ch].at[pl.ds(slot * W, CHUNK)],
                sem_slide,
            )

    def wait_slide():
        for ch in range(3):
            pltpu.make_async_copy(
                x_hbm.at[pl.ds(0, CHUNK)], ring[ch].at[pl.ds(0, CHUNK)], sem_slide
            ).wait()

    def issue_prefetch(c, pp):
        c4 = c * CR
        b = jnp.where(c4 <= H - CR, c4, H - CR) * W
        pltpu.async_copy(flow_hbm.at[pl.ds(fx_off + b, CHUNK)], fx_b[pp], sem_f)
        pltpu.async_copy(flow_hbm.at[pl.ds(fy_off + b, CHUNK)], fy_b[pp], sem_f)
        for ch in range(3):
            pltpu.async_copy(
                x_hbm.at[pl.ds(cur_ch_base[ch] + b, CHUNK)], xc_b[pp][ch], sem_xc
            )

    def wait_prefetch(pp):
        pltpu.make_async_copy(
            flow_hbm.at[pl.ds(0, CHUNK)], fx_b[pp], sem_f
        ).wait()
        pltpu.make_async_copy(
            flow_hbm.at[pl.ds(0, CHUNK)], fy_b[pp], sem_f
        ).wait()
        for ch in range(3):
            pltpu.make_async_copy(
                x_hbm.at[pl.ds(0, CHUNK)], xc_b[pp][ch], sem_xc
            ).wait()

    def do_chunk(c, pp, first=False):
        c4 = c * CR
        lo = jnp.where(c4 >= 40, c4 - 40, 0)
        hi = jnp.where(c4 + 47 <= H - 1, c4 + 47, H - 1)
        lo_f = lo.astype(jnp.float32)
        hi_f = hi.astype(jnp.float32)
        row0_f = c4.astype(jnp.float32)

        issue_slide(c)
        issue_prefetch(c + 1, (pp + 1) % 2)

        if not first:

            @pl.when(c >= 2)
            def _():
                for ch in range(3):
                    pltpu.make_async_copy(
                        out_b[pp][ch],
                        out_hbm.at[pl.ds(out_ch_base[ch] + c4 * W, CHUNK)],
                        sem_st[pp],
                    ).wait()

        def row_body(r, _):
            row_f = row0_f + r.astype(jnp.float32)
            roff = r * W
            mn0 = jnp.full((16,), float(H), jnp.float32)
            mx0 = jnp.full((16,), -1.0, jnp.float32)

            @plsc.parallel_loop(0, W // 16, carry=(mn0, mx0), unroll=4)
            def grp(k, carry):
                rx_mn, rx_mx = carry
                off = roff + k * 16
                fxv = fx_b[pp][pl.ds(off, 16)]
                fyv = fy_b[pp][pl.ds(off, 16)]
                rx = jnp.minimum(jnp.maximum(fxv + row_f, 0.0), float(H - 1))
                rx = (rx + MAGIC) - MAGIC
                colv = iota_f + (k * 16).astype(jnp.float32)
                ry = jnp.minimum(jnp.maximum(fyv + colv, 0.0), float(W - 1))
                ry = (ry + MAGIC) - MAGIC
                ryi = ry.astype(jnp.int32)
                rxc = jnp.minimum(jnp.maximum(rx, lo_f), hi_f).astype(jnp.int32)
                q = (rxc * MODM) >> 22
                addr = (rxc - q * RING) * W + ryi
                for ch in range(3):
                    g = plsc.load_gather(ring[ch], [addr])
                    xv = xc_b[pp][ch][pl.ds(off, 16)]
                    out_b[pp][ch][pl.ds(off, 16)] = xv - g
                return (jnp.minimum(rx_mn, rx), jnp.maximum(rx_mx, rx))

            rx_mn, rx_mx = grp

            row_bad = jnp.logical_or(
                jnp.min(rx_mn) < lo_f, jnp.max(rx_mx) > hi_f
            )

            @pl.when(row_bad)
            def _():
                def fix_k(kk, _):
                    o = roff + kk * 16
                    fxv = fx_b[pp][pl.ds(o, 16)]
                    fyv = fy_b[pp][pl.ds(o, 16)]
                    rx = jnp.minimum(jnp.maximum(fxv + row_f, 0.0), float(H - 1))
                    rx = (rx + MAGIC) - MAGIC
                    colv = iota_f + (kk * 16).astype(jnp.float32)
                    ry = jnp.minimum(jnp.maximum(fyv + colv, 0.0), float(W - 1))
                    ry = (ry + MAGIC) - MAGIC
                    fixidx[pl.ds(kk * 16, 16)] = (rx * float(W) + ry).astype(
                        jnp.int32
                    )
                    return 0

                lax.fori_loop(0, W // 16, fix_k, 0)
                for ch in range(3):
                    pltpu.async_copy(
                        x_hbm.at[pl.ds(src_ch_base[ch], HW)].at[fixidx],
                        out_b[pp][ch].at[pl.ds(roff, W)],
                        sem_fix,
                    )
                for ch in range(3):
                    pltpu.make_async_copy(
                        x_hbm.at[pl.ds(0, W)],
                        out_b[pp][ch].at[pl.ds(roff, W)],
                        sem_fix,
                    ).wait()

                def fix_sub(kk, _):
                    o = roff + kk * 16
                    for ch in range(3):
                        out_b[pp][ch][pl.ds(o, 16)] = (
                            xc_b[pp][ch][pl.ds(o, 16)]
                            - out_b[pp][ch][pl.ds(o, 16)]
                        )
                    return 0

                lax.fori_loop(0, W // 16, fix_sub, 0)

            return 0

        lax.fori_loop(0, CR, row_body, 0)


        for ch in range(3):
            pltpu.async_copy(
                out_b[pp][ch],
                out_hbm.at[pl.ds(out_ch_base[ch] + c4 * W, CHUNK)],
                sem_st[pp],
            )

    for ch in range(3):
        pltpu.async_copy(
            x_hbm.at[pl.ds(src_ch_base[ch], PRIME_ROWS * W)],
            ring[ch].at[pl.ds(0, PRIME_ROWS * W)],
            sem_slide,
        )
    issue_prefetch(jnp.int32(0), 0)
    for ch in range(3):
        pltpu.make_async_copy(
            x_hbm.at[pl.ds(0, PRIME_ROWS * W)],
            ring[ch].at[pl.ds(0, PRIME_ROWS * W)],
            sem_slide,
        ).wait()
    wait_prefetch(0)

    do_chunk(jnp.int32(0), 0, first=True)

    def super_body(s, _):
        c = 2 * s + 1
        wait_slide()
        wait_prefetch(1)
        do_chunk(c, 1)
        wait_slide()
        wait_prefetch(0)
        do_chunk(c + 1, 0)
        return 0

    lax.fori_loop(0, (NCHUNK - 2) // 2, super_body, 0)

    wait_slide()
    wait_prefetch(1)
    do_chunk(jnp.int32(NCHUNK - 1), 1)
    wait_slide()
    wait_prefetch(0)

    for pp in (0, 1):
        c4 = (NCHUNK - 2 + pp) * CR
        for ch in range(3):
            pltpu.make_async_copy(
                out_b[pp][ch],
                out_hbm.at[pl.ds(out_ch_base[ch] + c4 * W, CHUNK)],
                sem_st[pp],
            ).wait()


@jax.jit
def kernel(x, flow):
    x_flat = x.reshape(-1)
    flow_flat = flow.reshape(-1)

    mesh = plsc.VectorSubcoreMesh(core_axis_name="c", subcore_axis_name="s")
    out = pl.kernel(
        _fire_body,
        out_type=jax.ShapeDtypeStruct((NFRAMES * 6 * HW,), jnp.float32),
        mesh=mesh,
        compiler_params=pltpu.CompilerParams(needs_layout_passes=False),
        scratch_types=[
            pltpu.VMEM((RB,), jnp.float32),
            pltpu.VMEM((RB,), jnp.float32),
            pltpu.VMEM((RB,), jnp.float32),
            pltpu.VMEM((CHUNK,), jnp.float32),
            pltpu.VMEM((CHUNK,), jnp.float32),
            pltpu.VMEM((CHUNK,), jnp.float32),
            pltpu.VMEM((CHUNK,), jnp.float32),
            pltpu.VMEM((CHUNK,), jnp.float32),
            pltpu.VMEM((CHUNK,), jnp.float32),
            pltpu.VMEM((CHUNK,), jnp.float32),
            pltpu.VMEM((CHUNK,), jnp.float32),
            pltpu.VMEM((CHUNK,), jnp.float32),
            pltpu.VMEM((CHUNK,), jnp.float32),
            pltpu.VMEM((CHUNK,), jnp.float32),
            pltpu.VMEM((CHUNK,), jnp.float32),
            pltpu.VMEM((CHUNK,), jnp.float32),
            pltpu.VMEM((CHUNK,), jnp.float32),
            pltpu.VMEM((CHUNK,), jnp.float32),
            pltpu.VMEM((CHUNK,), jnp.float32),
            pltpu.VMEM((W,), jnp.int32),
            pltpu.SemaphoreType.DMA,
            pltpu.SemaphoreType.DMA,
            pltpu.SemaphoreType.DMA,
            pltpu.SemaphoreType.DMA,
            pltpu.SemaphoreType.DMA,
            pltpu.SemaphoreType.DMA,
        ],
    )(x_flat, flow_flat)
    return out.reshape(NFRAMES, 6, H, W)


# speedup vs baseline: 4.8155x; 1.0009x over previous
import jax
import jax.numpy as jnp
from jax import lax
from jax.experimental import pallas as pl
from jax.experimental.pallas import tpu as pltpu
from jax.experimental.pallas import tpu_sc as plsc

H = 384
W = 384
HW = H * W
NFRAMES = 16
CR = 4
CHUNK = CR * W
NCHUNK = H // CR
RING = 92
RB = RING * W
MODM = 45591
MAGIC = 8388608.0
PRIME_ROWS = 52


def _fire_body(x_hbm, flow_hbm, out_hbm, *sc):
    ring = sc[0:3]
    fx_b = sc[3:5]
    fy_b = sc[5:7]
    xc_b = (sc[7:10], sc[10:13])
    out_b = (sc[13:16], sc[16:19])
    fixidx = sc[19]
    sem_slide, sem_f, sem_xc, sem_st0, sem_st1, sem_fix = sc[20:26]
    sem_st = (sem_st0, sem_st1)

    cid = lax.axis_index("c")
    sid = lax.axis_index("s")
    wid = sid * 2 + cid
    tm1 = wid // 2
    d = wid % 2
    f = jnp.where(d == 0, tm1, 31 - tm1)
    src_t = jnp.where(d == 0, tm1 + 2, tm1)
    cur_t = tm1 + 1
    out_frame = tm1 * 6 + d * 3
    src_ch_base = [(src_t * 3 + ch) * HW for ch in range(3)]
    cur_ch_base = [(cur_t * 3 + ch) * HW for ch in range(3)]
    out_ch_base = [(out_frame + ch) * HW for ch in range(3)]
    fx_off = 2 * f * HW
    fy_off = (2 * f + 1) * HW

    iota_f = lax.iota(jnp.int32, 16).astype(jnp.float32)

    def issue_slide(c):
        c4 = c * CR
        row = jnp.where(c4 + 48 <= H - CR, c4 + 48, H - CR)
        slot = lax.rem(row, RING)
        for ch in range(3):
            pltpu.async_copy(
                x_hbm.at[pl.ds(src_ch_base[ch] + row * W, CHUNK)],
                ring[ch].at[pl.ds(slot * W, CHUNK)],
                sem_slide,
            )

    def wait_slide():
        for ch in range(3):
            pltpu.make_async_copy(
                x_hbm.at[pl.ds(0, CHUNK)], ring[ch].at[pl.ds(0, CHUNK)], sem_slide
            ).wait()

    def issue_prefetch(c, pp):
        c4 = c * CR
        b = jnp.where(c4 <= H - CR, c4, H - CR) * W
        pltpu.async_copy(flow_hbm.at[pl.ds(fx_off + b, CHUNK)], fx_b[pp], sem_f)
        pltpu.async_copy(flow_hbm.at[pl.ds(fy_off + b, CHUNK)], fy_b[pp], sem_f)
        for ch in range(3):
            pltpu.async_copy(
                x_hbm.at[pl.ds(cur_ch_base[ch] + b, CHUNK)], xc_b[pp][ch], sem_xc
            )

    def wait_prefetch(pp):
        pltpu.make_async_copy(
            flow_hbm.at[pl.ds(0, CHUNK)], fx_b[pp], sem_f
        ).wait()
        pltpu.make_async_copy(
            flow_hbm.at[pl.ds(0, CHUNK)], fy_b[pp], sem_f
        ).wait()
        for ch in range(3):
            pltpu.make_async_copy(
                x_hbm.at[pl.ds(0, CHUNK)], xc_b[pp][ch], sem_xc
            ).wait()

    def do_chunk(c, pp, first=False):
        c4 = c * CR
        lo = jnp.where(c4 >= 40, c4 - 40, 0)
        hi = jnp.where(c4 + 47 <= H - 1, c4 + 47, H - 1)
        lo_f = lo.astype(jnp.float32)
        hi_f = hi.astype(jnp.float32)
        row0_f = c4.astype(jnp.float32)

        issue_slide(c)
        issue_prefetch(c + 1, (pp + 1) % 2)

        if not first:

            @pl.when(c >= 2)
            def _():
                for ch in range(3):
                    pltpu.make_async_copy(
                        out_b[pp][ch],
                        out_hbm.at[pl.ds(out_ch_base[ch] + c4 * W, CHUNK)],
                        sem_st[pp],
                    ).wait()

        def row_body(r, _):
            row_f = row0_f + r.astype(jnp.float32)
            roff = r * W
            mn0 = jnp.full((16,), float(H), jnp.float32)
            mx0 = jnp.full((16,), -1.0, jnp.float32)

            @plsc.parallel_loop(0, W // 16, carry=(mn0, mx0), unroll=6)
            def grp(k, carry):
                rx_mn, rx_mx = carry
                off = roff + k * 16
                fxv = fx_b[pp][pl.ds(off, 16)]
                fyv = fy_b[pp][pl.ds(off, 16)]
                rx = jnp.minimum(jnp.maximum(fxv + row_f, 0.0), float(H - 1))
                rx = (rx + MAGIC) - MAGIC
                colv = iota_f + (k * 16).astype(jnp.float32)
                ry = jnp.minimum(jnp.maximum(fyv + colv, 0.0), float(W - 1))
                ry = (ry + MAGIC) - MAGIC
                ryi = ry.astype(jnp.int32)
                rxc = jnp.minimum(jnp.maximum(rx, lo_f), hi_f).astype(jnp.int32)
                q = (rxc * MODM) >> 22
                addr = (rxc - q * RING) * W + ryi
                for ch in range(3):
                    g = plsc.load_gather(ring[ch], [addr])
                    xv = xc_b[pp][ch][pl.ds(off, 16)]
                    out_b[pp][ch][pl.ds(off, 16)] = xv - g
                return (jnp.minimum(rx_mn, rx), jnp.maximum(rx_mx, rx))

            rx_mn, rx_mx = grp

            row_bad = jnp.logical_or(
                jnp.min(rx_mn) < lo_f, jnp.max(rx_mx) > hi_f
            )

            @pl.when(row_bad)
            def _():
                def fix_k(kk, _):
                    o = roff + kk * 16
                    fxv = fx_b[pp][pl.ds(o, 16)]
                    fyv = fy_b[pp][pl.ds(o, 16)]
                    rx = jnp.minimum(jnp.maximum(fxv + row_f, 0.0), float(H - 1))
                    rx = (rx + MAGIC) - MAGIC
                    colv = iota_f + (kk * 16).astype(jnp.float32)
                    ry = jnp.minimum(jnp.maximum(fyv + colv, 0.0), float(W - 1))
                    ry = (ry + MAGIC) - MAGIC
                    fixidx[pl.ds(kk * 16, 16)] = (rx * float(W) + ry).astype(
                        jnp.int32
                    )
                    return 0

                lax.fori_loop(0, W // 16, fix_k, 0)
                for ch in range(3):
                    pltpu.async_copy(
                        x_hbm.at[pl.ds(src_ch_base[ch], HW)].at[fixidx],
                        out_b[pp][ch].at[pl.ds(roff, W)],
                        sem_fix,
                    )
                for ch in range(3):
                    pltpu.make_async_copy(
                        x_hbm.at[pl.ds(0, W)],
                        out_b[pp][ch].at[pl.ds(roff, W)],
                        sem_fix,
                    ).wait()

                def fix_sub(kk, _):
                    o = roff + kk * 16
                    for ch in range(3):
                        out_b[pp][ch][pl.ds(o, 16)] = (
                            xc_b[pp][ch][pl.ds(o, 16)]
                            - out_b[pp][ch][pl.ds(o, 16)]
                        )
                    return 0

                lax.fori_loop(0, W // 16, fix_sub, 0)

            return 0

        lax.fori_loop(0, CR, row_body, 0)


        for ch in range(3):
            pltpu.async_copy(
                out_b[pp][ch],
                out_hbm.at[pl.ds(out_ch_base[ch] + c4 * W, CHUNK)],
                sem_st[pp],
            )

    for ch in range(3):
        pltpu.async_copy(
            x_hbm.at[pl.ds(src_ch_base[ch], PRIME_ROWS * W)],
            ring[ch].at[pl.ds(0, PRIME_ROWS * W)],
            sem_slide,
        )
    issue_prefetch(jnp.int32(0), 0)
    for ch in range(3):
        pltpu.make_async_copy(
            x_hbm.at[pl.ds(0, PRIME_ROWS * W)],
            ring[ch].at[pl.ds(0, PRIME_ROWS * W)],
            sem_slide,
        ).wait()
    wait_prefetch(0)

    do_chunk(jnp.int32(0), 0, first=True)

    def super_body(s, _):
        c = 2 * s + 1
        wait_slide()
        wait_prefetch(1)
        do_chunk(c, 1)
        wait_slide()
        wait_prefetch(0)
        do_chunk(c + 1, 0)
        return 0

    lax.fori_loop(0, (NCHUNK - 2) // 2, super_body, 0)

    wait_slide()
    wait_prefetch(1)
    do_chunk(jnp.int32(NCHUNK - 1), 1)
    wait_slide()
    wait_prefetch(0)

    for pp in (0, 1):
        c4 = (NCHUNK - 2 + pp) * CR
        for ch in range(3):
            pltpu.make_async_copy(
                out_b[pp][ch],
                out_hbm.at[pl.ds(out_ch_base[ch] + c4 * W, CHUNK)],
                sem_st[pp],
            ).wait()


@jax.jit
def kernel(x, flow):
    x_flat = x.reshape(-1)
    flow_flat = flow.reshape(-1)

    mesh = plsc.VectorSubcoreMesh(core_axis_name="c", subcore_axis_name="s")
    out = pl.kernel(
        _fire_body,
        out_type=jax.ShapeDtypeStruct((NFRAMES * 6 * HW,), jnp.float32),
        mesh=mesh,
        compiler_params=pltpu.CompilerParams(needs_layout_passes=False),
        scratch_types=[
            pltpu.VMEM((RB,), jnp.float32),
            pltpu.VMEM((RB,), jnp.float32),
            pltpu.VMEM((RB,), jnp.float32),
            pltpu.VMEM((CHUNK,), jnp.float32),
            pltpu.VMEM((CHUNK,), jnp.float32),
            pltpu.VMEM((CHUNK,), jnp.float32),
            pltpu.VMEM((CHUNK,), jnp.float32),
            pltpu.VMEM((CHUNK,), jnp.float32),
            pltpu.VMEM((CHUNK,), jnp.float32),
            pltpu.VMEM((CHUNK,), jnp.float32),
            pltpu.VMEM((CHUNK,), jnp.float32),
            pltpu.VMEM((CHUNK,), jnp.float32),
            pltpu.VMEM((CHUNK,), jnp.float32),
            pltpu.VMEM((CHUNK,), jnp.float32),
            pltpu.VMEM((CHUNK,), jnp.float32),
            pltpu.VMEM((CHUNK,), jnp.float32),
            pltpu.VMEM((CHUNK,), jnp.float32),
            pltpu.VMEM((CHUNK,), jnp.float32),
            pltpu.VMEM((CHUNK,), jnp.float32),
            pltpu.VMEM((W,), jnp.int32),
            pltpu.SemaphoreType.DMA,
            pltpu.SemaphoreType.DMA,
            pltpu.SemaphoreType.DMA,
            pltpu.SemaphoreType.DMA,
            pltpu.SemaphoreType.DMA,
            pltpu.SemaphoreType.DMA,
        ],
    )(x_flat, flow_flat)
    return out.reshape(NFRAMES, 6, H, W)
